# Initial kernel scaffold; baseline (speedup 1.0000x reference)
import jax
import jax.numpy as jnp
from jax.experimental import pallas as pl


def kernel(x, edge_index, W0, Wm, Ws, Wp, Wa, Dm, bm, Ds, bs, Dp, bp, Da, ba):
    raise NotImplementedError("write your pallas kernel here")


# speedup vs baseline: 8.2525x; 8.2525x over previous
import functools

import jax
import jax.numpy as jnp
from jax import lax
from jax.experimental import pallas as pl
from jax.experimental.pallas import tpu as pltpu
from jax.experimental.pallas import tpu_sc as plsc

SMALL = 1e-16
CH = 128
NS = 16
NC = 2
BM = 512
KC = 16


def _sc_mesh():
    return plsc.VectorSubcoreMesh(core_axis_name="c", subcore_axis_name="s")


@functools.lru_cache(maxsize=None)
def _deg_sc(n_pad, n_ops):
    def body(src_hbm, ones_hbm, z_hbm, deg_o, src_v, ones_v, acc, sem):
        c = lax.axis_index("c")
        s = lax.axis_index("s")
        rps = n_pad // NS

        @pl.when(c == 0)
        def _():
            pltpu.sync_copy(src_hbm.at[pl.ds(s * n_ops, n_ops)], src_v)
            pltpu.sync_copy(ones_hbm, ones_v)
            pltpu.sync_copy(z_hbm.at[pl.ds(s * rps, rps)],
                            acc.at[pl.ds(s * rps, rps)])
            plsc.subcore_barrier()

            k = 8

            def group(g, carry):
                base = g * k
                for t in range(k):
                    pltpu.async_copy(ones_v, acc.at[src_v.at[base + t]],
                                     sem, add=True)
                for t in range(k):
                    pltpu.make_async_copy(ones_v, acc.at[src_v.at[base + t]],
                                          sem).wait()
                return carry

            lax.fori_loop(0, n_ops // k, group, 0)
            for j in range((n_ops // k) * k, n_ops):
                pltpu.sync_copy(ones_v, acc.at[src_v.at[j]], add=True)

            plsc.subcore_barrier()
            pltpu.sync_copy(acc.at[pl.ds(s * rps, rps)],
                            deg_o.at[pl.ds(s * rps, rps)])

    return pl.kernel(
        body,
        out_type=[jax.ShapeDtypeStruct((n_pad,), jnp.float32)],
        mesh=_sc_mesh(),
        scratch_types=[
            pltpu.VMEM((n_ops, CH), jnp.int32),
            pltpu.VMEM((CH,), jnp.float32),
            pltpu.VMEM_SHARED((n_pad,), jnp.float32),
            pltpu.SemaphoreType.DMA,
        ],
    )


def _spmm_chunk_loop(t_of, src_hbm, dst_hbm, acc, src_c, dst_c,
                     bufa, bufb, sa, sb, row0, n_my):
    def fire(j, buf, sem):
        pltpu.async_copy(t_of.at[src_c.at[j]], buf, sem)

    def wait(j, buf, sem):
        pltpu.make_async_copy(t_of.at[src_c.at[j]], buf, sem).wait()

    def chunk(q, carry):
        off = row0 + q * KC
        pltpu.sync_copy(src_hbm.at[pl.ds(off, KC)], src_c)
        pltpu.sync_copy(dst_hbm.at[pl.ds(off, KC)], dst_c)
        fire(0, bufa, sa)

        def step(h, carry2):
            j = 2 * h
            wait(j, bufa, sa)
            fire(j + 1, bufb, sb)
            pltpu.sync_copy(bufa, acc.at[dst_c.at[j]], add=True)
            wait(j + 1, bufb, sb)

            @pl.when(j + 2 < KC)
            def _():
                fire(j + 2, bufa, sa)

            pltpu.sync_copy(bufb, acc.at[dst_c.at[j + 1]], add=True)
            return carry2

        lax.fori_loop(0, KC // 2, step, 0)
        return carry

    lax.fori_loop(0, n_my // KC, chunk, 0)


@functools.lru_cache(maxsize=None)
def _spmm_es_sc(n_pad, n_ops):
    def body(src_hbm, dst_hbm, t, z_hbm, o0, o1,
             src_c, dst_c, bufa, bufb, acc, sa, sb):
        c = lax.axis_index("c")
        s = lax.axis_index("s")
        rps = n_pad // NS
        n_half = n_ops // 2

        pltpu.sync_copy(z_hbm.at[pl.ds(s * rps, rps)],
                        acc.at[pl.ds(s * rps, rps)])
        plsc.subcore_barrier()

        _spmm_chunk_loop(t, src_hbm, dst_hbm, acc, src_c, dst_c,
                         bufa, bufb, sa, sb,
                         s * n_ops + c * n_half, n_half)
        plsc.subcore_barrier()

        @pl.when(c == 0)
        def _():
            pltpu.sync_copy(acc.at[pl.ds(s * rps, rps)],
                            o0.at[pl.ds(s * rps, rps)])

        @pl.when(c == 1)
        def _():
            pltpu.sync_copy(acc.at[pl.ds(s * rps, rps)],
                            o1.at[pl.ds(s * rps, rps)])

    return pl.kernel(
        body,
        out_type=[jax.ShapeDtypeStruct((n_pad, 128), jnp.float32)] * 2,
        mesh=_sc_mesh(),
        scratch_types=[
            pltpu.VMEM((KC, CH), jnp.int32),
            pltpu.VMEM((KC, CH), jnp.int32),
            pltpu.VMEM((CH, 128), jnp.float32),
            pltpu.VMEM((CH, 128), jnp.float32),
            pltpu.VMEM_SHARED((n_pad, 128), jnp.float32),
            pltpu.SemaphoreType.DMA,
            pltpu.SemaphoreType.DMA,
        ],
    )


@functools.lru_cache(maxsize=None)
def _spmm_cs_sc(n_pad, n_ops, fh):
    def body(src_hbm, dst_hbm, t0, t1, z_hbm, o0, o1,
             src_c, dst_c, bufa, bufb, acc, sa, sb):
        c = lax.axis_index("c")
        s = lax.axis_index("s")
        rps = n_pad // NS

        pltpu.sync_copy(z_hbm.at[pl.ds(s * rps, rps)],
                        acc.at[pl.ds(s * rps, rps)])
        plsc.subcore_barrier()

        @pl.when(c == 0)
        def _():
            _spmm_chunk_loop(t0, src_hbm, dst_hbm, acc, src_c, dst_c,
                             bufa, bufb, sa, sb, s * n_ops, n_ops)

        @pl.when(c == 1)
        def _():
            _spmm_chunk_loop(t1, src_hbm, dst_hbm, acc, src_c, dst_c,
                             bufa, bufb, sa, sb, s * n_ops, n_ops)

        plsc.subcore_barrier()

        @pl.when(c == 0)
        def _():
            pltpu.sync_copy(acc.at[pl.ds(s * rps, rps)],
                            o0.at[pl.ds(s * rps, rps)])

        @pl.when(c == 1)
        def _():
            pltpu.sync_copy(acc.at[pl.ds(s * rps, rps)],
                            o1.at[pl.ds(s * rps, rps)])

    return pl.kernel(
        body,
        out_type=[jax.ShapeDtypeStruct((n_pad, fh), jnp.float32)] * 2,
        mesh=_sc_mesh(),
        scratch_types=[
            pltpu.VMEM((KC, CH), jnp.int32),
            pltpu.VMEM((KC, CH), jnp.int32),
            pltpu.VMEM((CH, fh), jnp.float32),
            pltpu.VMEM((CH, fh), jnp.float32),
            pltpu.VMEM_SHARED((n_pad, fh), jnp.float32),
            pltpu.SemaphoreType.DMA,
            pltpu.SemaphoreType.DMA,
        ],
    )


def _mm1_body(x_ref, w_ref, d_ref, o_ref):
    r = lax.rsqrt(d_ref[...] + SMALL)
    xw = jnp.dot(x_ref[...], w_ref[...], preferred_element_type=jnp.float32)
    o_ref[...] = xw * r


def _mm2_body(h0_ref, h1_ref, d_ref, w_ref, o0_ref, o1_ref):
    r = lax.rsqrt(d_ref[...] + SMALL)
    h = (h0_ref[...] + h1_ref[...]) * r
    hw = jnp.dot(h, w_ref[...], preferred_element_type=jnp.float32) * r
    o0_ref[...] = hw[:, :128]
    o1_ref[...] = hw[:, 128:]


def _mm3_body(a0_ref, a1_ref, d_ref, wm_ref, ws_ref, wp_ref, wa_ref,
              bm_ref, bs_ref, bp_ref, ba_ref,
              om_ref, os_ref, op_ref, oa_ref):
    r = lax.rsqrt(d_ref[...] + SMALL)
    a = jnp.concatenate([a0_ref[...], a1_ref[...]], axis=1) * r
    sg = jax.nn.sigmoid(a)
    om_ref[...] = jnp.dot(sg[:, 0:64], wm_ref[...],
                          preferred_element_type=jnp.float32) + bm_ref[...]
    os_ref[...] = jnp.dot(sg[:, 64:128], ws_ref[...],
                          preferred_element_type=jnp.float32) + bs_ref[...]
    op_ref[...] = jnp.dot(sg[:, 128:192], wp_ref[...],
                          preferred_element_type=jnp.float32) + bp_ref[...]
    t = jnp.dot(sg[:, 192:256], wa_ref[...],
                preferred_element_type=jnp.float32) + ba_ref[...]
    oa_ref[...] = jnp.maximum(t, 0.0) + jnp.log1p(jnp.exp(-jnp.abs(t)))


def _row_spec(cols):
    return pl.BlockSpec((BM, cols), lambda i: (i, 0))


def _full_spec(shape):
    return pl.BlockSpec(shape, lambda i: (0,) * len(shape))


@functools.lru_cache(maxsize=None)
def _mm1(n_pad):
    return pl.pallas_call(
        _mm1_body,
        grid=(n_pad // BM,),
        in_specs=[_row_spec(128), _full_spec((128, 128)), _row_spec(1)],
        out_specs=_row_spec(128),
        out_shape=jax.ShapeDtypeStruct((n_pad, 128), jnp.float32),
        compiler_params=pltpu.CompilerParams(
            dimension_semantics=("parallel",)),
    )


@functools.lru_cache(maxsize=None)
def _mm2(n_pad):
    return pl.pallas_call(
        _mm2_body,
        grid=(n_pad // BM,),
        in_specs=[_row_spec(128), _row_spec(128), _row_spec(1),
                  _full_spec((128, 256))],
        out_specs=[_row_spec(128), _row_spec(128)],
        out_shape=[jax.ShapeDtypeStruct((n_pad, 128), jnp.float32)] * 2,
        compiler_params=pltpu.CompilerParams(
            dimension_semantics=("parallel",)),
    )


@functools.lru_cache(maxsize=None)
def _mm3(n_pad):
    return pl.pallas_call(
        _mm3_body,
        grid=(n_pad // BM,),
        in_specs=[_row_spec(128), _row_spec(128), _row_spec(1)]
        + [_full_spec((64, 64))] * 4 + [_full_spec((1, 64))] * 4,
        out_specs=[_row_spec(64)] * 4,
        out_shape=[jax.ShapeDtypeStruct((n_pad, 64), jnp.float32)] * 4,
        compiler_params=pltpu.CompilerParams(
            dimension_semantics=("parallel",)),
    )


def kernel(x, edge_index, W0, Wm, Ws, Wp, Wa, Dm, bm, Ds, bs, Dp, bp, Da, ba):
    n, d_feat = x.shape
    e = edge_index.shape[1]

    n_pad = ((n + 2047) // 2048) * 2048
    e_blk = NS * CH
    n_ops = -(-e // e_blk)
    n_ops = ((n_ops + 2 * KC - 1) // (2 * KC)) * (2 * KC)
    e_pad = n_ops * e_blk

    src = edge_index[0].astype(jnp.int32)
    dst = edge_index[1].astype(jnp.int32)
    fill = jnp.full((e_pad - e,), n, jnp.int32)
    src_p = jnp.concatenate([src, fill]).reshape(NS * n_ops, CH)
    dst_p = jnp.concatenate([dst, fill]).reshape(NS * n_ops, CH)

    x_pad = jnp.zeros((n_pad, d_feat), jnp.float32).at[:n].set(
        x.astype(jnp.float32))
    ones = jnp.ones((CH,), jnp.float32)
    z1 = jnp.zeros((n_pad,), jnp.float32)
    z128 = jnp.zeros((n_pad, 128), jnp.float32)
    wcat = jnp.concatenate([Wm, Ws, Wp, Wa], axis=1)

    (deg,) = _deg_sc(n_pad, n_ops)(src_p, ones, z1)
    deg2 = deg.reshape(n_pad, 1)

    t = _mm1(n_pad)(x_pad, W0, deg2)
    h0, h1 = _spmm_es_sc(n_pad, n_ops)(src_p, dst_p, t, z128)
    g0, g1 = _mm2(n_pad)(h0, h1, deg2, wcat)
    a0, a1 = _spmm_cs_sc(n_pad, n_ops, 128)(src_p, dst_p, g0, g1, z128)
    zm, zs, zp, za = _mm3(n_pad)(
        a0, a1, deg2, Dm, Ds, Dp, Da,
        bm.reshape(1, 64), bs.reshape(1, 64),
        bp.reshape(1, 64), ba.reshape(1, 64))

    return (zm[:n], zs[:n], zp[:n], za[:n])
